# baseline (speedup 1.0000x reference)
import jax
import jax.numpy as jnp
from jax import lax
from jax.experimental import pallas as pl
from jax.experimental.pallas import tpu as pltpu
from jax.experimental.pallas import tpu_sc as plsc

EMB_DIM = 64
NUM_NEG = 5
NC = 2
NS = 16
NW = NC * NS
LANES = 16
PER_TILE = 512
CHUNK = 64
GROUPS = CHUNK // LANES
NROWS = CHUNK * NUM_NEG

_LN2 = 0.6931471805599453


def _softplus(x):
    t = jnp.exp(x)
    y = 1.0 + t
    b = lax.bitcast_convert_type(y, jnp.int32)
    e = (b >> 23) - 127
    m = lax.bitcast_convert_type((b & 0x007FFFFF) | 0x3F800000, jnp.float32)
    z = (m - 1.0) / (m + 1.0)
    z2 = z * z
    p = z * (2.0 + z2 * (0.66666667 + z2 * (0.4 + z2 * (0.28571429 + z2 * 0.22222222))))
    return e.astype(jnp.float32) * _LN2 + p


def _body(pos_u, pos_v, neg_f, u_emb, v2, out,
          pu_idx, pv_idx, ng_idx, pv_half, ng_half,
          u_buf0, u_buf1, v_buf0, v_buf1, n_buf0, n_buf1,
          acc_buf, sem0, sem1):
    wid = lax.axis_index("s") * NC + lax.axis_index("c")
    base = wid * PER_TILE

    pltpu.sync_copy(pos_u.at[pl.ds(base, PER_TILE)], pu_idx)
    pltpu.sync_copy(pos_v.at[pl.ds(base, PER_TILE)], pv_idx)
    pltpu.sync_copy(neg_f.at[pl.ds(base * NUM_NEG, PER_TILE * NUM_NEG)], ng_idx)

    def halve(i, _):
        pv_half[pl.ds(i * LANES, LANES)] = pv_idx[pl.ds(i * LANES, LANES)] >> 1
        return 0
    lax.fori_loop(0, PER_TILE // LANES, halve, 0)

    def halve_n(i, _):
        ng_half[pl.ds(i * LANES, LANES)] = ng_idx[pl.ds(i * LANES, LANES)] >> 1
        return 0
    lax.fori_loop(0, PER_TILE * NUM_NEG // LANES, halve_n, 0)

    u_bufs = (u_buf0, u_buf1)
    v_bufs = (v_buf0, v_buf1)
    n_bufs = (n_buf0, n_buf1)
    sems = (sem0, sem1)

    def fire(c, slot):
        ub, vb, nb = u_bufs[slot], v_bufs[slot], n_bufs[slot]
        sem = sems[slot]

        pltpu.async_copy(v2.at[pv_half.at[pl.ds(c * CHUNK, CHUNK)]], vb, sem)
        pltpu.async_copy(v2.at[ng_half.at[pl.ds(c * NROWS, 128)]],
                         nb.at[pl.ds(0, 128)], sem)
        pltpu.async_copy(v2.at[ng_half.at[pl.ds(c * NROWS + 128, 128)]],
                         nb.at[pl.ds(128, 128)], sem)
        pltpu.async_copy(v2.at[ng_half.at[pl.ds(c * NROWS + 256, 64)]],
                         nb.at[pl.ds(256, 64)], sem)

        def fire_u(g, _):
            uvec = pu_idx[pl.ds(c * CHUNK + g * LANES, LANES)]
            row0 = g * LANES
            for j in range(LANES):
                pltpu.async_copy(u_emb.at[uvec[j]], ub.at[row0 + j], sem)
            return 0

        lax.fori_loop(0, GROUPS, fire_u, 0)

    def drain(slot):
        pltpu.make_async_copy(u_emb.at[pl.ds(0, CHUNK)], u_bufs[slot], sems[slot]).wait()
        pltpu.make_async_copy(v2.at[pl.ds(0, CHUNK)], v_bufs[slot], sems[slot]).wait()
        pltpu.make_async_copy(v2.at[pl.ds(0, NROWS)], n_bufs[slot], sems[slot]).wait()

    lane_iota = lax.iota(jnp.int32, LANES)

    def compute(c, slot, acc):
        ub, vb, nb = u_bufs[slot], v_bufs[slot], n_bufs[slot]

        def group_step(g, acc):
            parv = pv_idx[pl.ds(c * CHUNK + g * LANES, LANES)] & 1
            base_n = c * NROWS + g * LANES * NUM_NEG
            parn = [
                (plsc.load_gather(
                    ng_idx, [base_n + lane_iota * NUM_NEG + n]) & 1)
                for n in range(NUM_NEG)
            ]

            def item_step(j, carry):
                sve, svo, ne, no = carry
                i = g * LANES + j
                u = [ub[i, pl.ds(k * LANES, LANES)] for k in range(4)]

                def dot2(ref, row):
                    pe = u[0] * ref[row, pl.ds(0, LANES)]
                    po = u[0] * ref[row, pl.ds(EMB_DIM, LANES)]
                    for k in range(1, 4):
                        pe = pe + u[k] * ref[row, pl.ds(k * LANES, LANES)]
                        po = po + u[k] * ref[row, pl.ds(EMB_DIM + k * LANES, LANES)]
                    return jnp.sum(pe), jnp.sum(po)

                msk = lane_iota == j
                se, so = dot2(vb, i)
                sve = jnp.where(msk, se, sve)
                svo = jnp.where(msk, so, svo)
                ne_out, no_out = [], []
                for n in range(NUM_NEG):
                    te, to = dot2(nb, i * NUM_NEG + n)
                    ne_out.append(jnp.where(msk, te, ne[n]))
                    no_out.append(jnp.where(msk, to, no[n]))
                return sve, svo, tuple(ne_out), tuple(no_out)

            z = jnp.zeros((LANES,), jnp.float32)
            z5 = (z, z, z, z, z)
            sve, svo, ne, no = lax.fori_loop(
                0, LANES, item_step, (z, z, z5, z5))

            s = jnp.where(parv == 1, svo, sve)
            acc = acc + _softplus(-jnp.clip(s, -10.0, 10.0))
            for n in range(NUM_NEG):
                t = jnp.where(parn[n] == 1, no[n], ne[n])
                acc = acc + _softplus(jnp.clip(t, -10.0, 10.0))
            return acc

        return lax.fori_loop(0, GROUPS, group_step, acc)

    n_chunks = PER_TILE // CHUNK
    acc = jnp.zeros((LANES,), jnp.float32)
    fire(0, 0)
    for c in range(n_chunks):
        if c + 1 < n_chunks:
            fire(c + 1, (c + 1) % 2)
        drain(c % 2)
        acc = compute(c, c % 2, acc)

    acc_buf[...] = acc
    pltpu.sync_copy(acc_buf, out.at[wid])


@jax.jit
def _sc_skipgram(pos_u, pos_v, neg_f, u_emb, v2):
    mesh = plsc.VectorSubcoreMesh(core_axis_name="c", subcore_axis_name="s")
    kcall = pl.kernel(
        _body,
        out_type=jax.ShapeDtypeStruct((NW, LANES), jnp.float32),
        mesh=mesh,
        compiler_params=pltpu.CompilerParams(needs_layout_passes=False),
        scratch_types=[
            pltpu.VMEM((PER_TILE,), jnp.int32),
            pltpu.VMEM((PER_TILE,), jnp.int32),
            pltpu.VMEM((PER_TILE * NUM_NEG,), jnp.int32),
            pltpu.VMEM((PER_TILE,), jnp.int32),
            pltpu.VMEM((PER_TILE * NUM_NEG,), jnp.int32),
            pltpu.VMEM((CHUNK, EMB_DIM), jnp.float32),
            pltpu.VMEM((CHUNK, EMB_DIM), jnp.float32),
            pltpu.VMEM((CHUNK, 2 * EMB_DIM), jnp.float32),
            pltpu.VMEM((CHUNK, 2 * EMB_DIM), jnp.float32),
            pltpu.VMEM((NROWS, 2 * EMB_DIM), jnp.float32),
            pltpu.VMEM((NROWS, 2 * EMB_DIM), jnp.float32),
            pltpu.VMEM((LANES,), jnp.float32),
            pltpu.SemaphoreType.DMA,
            pltpu.SemaphoreType.DMA,
        ],
    )
    return kcall(pos_u, pos_v, neg_f, u_emb, v2)


def kernel(pos_u, pos_v, neg_v, u_emb, v_emb):
    batch = pos_u.shape[0]
    neg_f = neg_v.astype(jnp.int32).reshape(-1)
    v2 = v_emb.reshape(v_emb.shape[0] // 2, 2 * EMB_DIM)
    partials = _sc_skipgram(pos_u.astype(jnp.int32), pos_v.astype(jnp.int32),
                            neg_f, u_emb, v2)
    return jnp.sum(partials) * (1.0 / batch)
